# baseline (speedup 1.0000x reference)
import functools

import jax
import jax.numpy as jnp
from jax import lax
from jax.experimental import pallas as pl
from jax.experimental.pallas import tpu as pltpu
from jax.experimental.pallas import tpu_sc as plsc

_NC = 2
_NS = 16
_NW = _NC * _NS
_L = 16

_N = 8388608
_PTS_PER_W = _N // _NW
_CHUNK = 16384
_NCHUNK = _PTS_PER_W // _CHUNK
_GROUPS = _CHUNK // _L


def _conf_body(lg_hbm, gt_hbm, out_hbm, lg_v, gt_v, res_v):
    cid = lax.axis_index("c")
    sid = lax.axis_index("s")
    wid = cid * _NS + sid
    base = wid * _PTS_PER_W

    lane = lax.iota(jnp.int32, _L)
    idx_even = lane * 2
    zeros = jnp.zeros((_L,), jnp.int32)
    ones = jnp.ones((_L,), jnp.int32)

    def chunk_body(c, accs):
        off = base + c * _CHUNK
        pltpu.sync_copy(lg_hbm.at[pl.ds(off * 2, _CHUNK * 2)], lg_v)
        pltpu.sync_copy(gt_hbm.at[pl.ds(off, _CHUNK)], gt_v)

        def group_body(j, accs2):
            a_p, a_g, a_pg = accs2
            gt16 = gt_v[pl.ds(j * _L, _L)]
            ia = j * (2 * _L) + idx_even
            l0 = plsc.load_gather(lg_v, [ia])
            l1 = plsc.load_gather(lg_v, [ia + 1])
            pred = l1 > l0
            a_p = a_p + jnp.where(pred, ones, zeros)
            a_g = a_g + gt16
            a_pg = a_pg + jnp.where(pred, gt16, zeros)
            return (a_p, a_g, a_pg)

        return lax.fori_loop(0, _GROUPS, group_body, accs)

    z = jnp.zeros((_L,), jnp.int32)
    acc_p, acc_g, acc_pg = lax.fori_loop(0, _NCHUNK, chunk_body, (z, z, z))
    res_v[pl.ds(0, _L)] = acc_p
    res_v[pl.ds(_L, _L)] = acc_g
    res_v[pl.ds(2 * _L, _L)] = acc_pg
    pltpu.sync_copy(res_v, out_hbm.at[pl.ds(wid * 3 * _L, 3 * _L)])


_conf = functools.partial(
    pl.kernel,
    mesh=plsc.VectorSubcoreMesh(core_axis_name="c", subcore_axis_name="s"),
    out_type=jax.ShapeDtypeStruct((_NW * 3 * _L,), jnp.int32),
    compiler_params=pltpu.CompilerParams(needs_layout_passes=False),
    scratch_types=[
        pltpu.VMEM((2 * _CHUNK,), jnp.float32),
        pltpu.VMEM((_CHUNK,), jnp.int32),
        pltpu.VMEM((3 * _L,), jnp.int32),
    ],
)(_conf_body)


def kernel(pred_logits, gt_labels):
    lg = pred_logits.reshape(-1)
    parts = _conf(lg, gt_labels)
    p = parts.reshape(_NW, 3, _L).sum(axis=(0, 2))
    s_p, s_g, s_pg = p[0], p[1], p[2]
    n = jnp.int32(pred_logits.shape[0])
    c00 = n - s_p - s_g + s_pg
    c01 = s_g - s_pg
    c10 = s_p - s_pg
    c11 = s_pg
    return jnp.stack([jnp.stack([c00, c01]), jnp.stack([c10, c11])]).astype(
        jnp.int32)


# speedup vs baseline: 1.0080x; 1.0080x over previous
import functools

import jax
import jax.numpy as jnp
from jax import lax
from jax.experimental import pallas as pl
from jax.experimental.pallas import tpu as pltpu
from jax.experimental.pallas import tpu_sc as plsc

_NC = 2
_NS = 16
_NW = _NC * _NS
_L = 16

_N = 8388608
_PTS_PER_W = _N // _NW
_CHUNK = 16384
_NCHUNK = _PTS_PER_W // _CHUNK
_GROUPS = _CHUNK // _L


def _conf_body(lg_hbm, gt_hbm, out_hbm,
               lg_v0, lg_v1, gt_v0, gt_v1, res_v,
               sem_lg0, sem_lg1, sem_gt0, sem_gt1):
    cid = lax.axis_index("c")
    sid = lax.axis_index("s")
    wid = cid * _NS + sid
    base = wid * _PTS_PER_W

    lane = lax.iota(jnp.int32, _L)
    idx_even = lane * 2
    zeros = jnp.zeros((_L,), jnp.int32)
    ones = jnp.ones((_L,), jnp.int32)

    lg_bufs = (lg_v0, lg_v1)
    gt_bufs = (gt_v0, gt_v1)
    sem_lg = (sem_lg0, sem_lg1)
    sem_gt = (sem_gt0, sem_gt1)

    def start(c, b):
        off = base + c * _CHUNK
        h1 = pltpu.async_copy(
            lg_hbm.at[pl.ds(off * 2, _CHUNK * 2)], lg_bufs[b], sem_lg[b])
        h2 = pltpu.async_copy(
            gt_hbm.at[pl.ds(off, _CHUNK)], gt_bufs[b], sem_gt[b])
        return h1, h2

    def group_body(lg_b, gt_b, j, accs2):
        a_p, a_g, a_pg = accs2
        gt16 = gt_b[pl.ds(j * _L, _L)]
        ia = j * (2 * _L) + idx_even
        l0 = plsc.load_gather(lg_b, [ia])
        l1 = plsc.load_gather(lg_b, [ia + 1])
        pred = l1 > l0
        a_p = a_p + jnp.where(pred, ones, zeros)
        a_g = a_g + gt16
        a_pg = a_pg + jnp.where(pred, gt16, zeros)
        return (a_p, a_g, a_pg)

    z = jnp.zeros((_L,), jnp.int32)
    accs = (z, z, z)
    handles = [None, None]
    handles[0] = start(0, 0)
    for c in range(_NCHUNK):
        b = c % 2
        if c + 1 < _NCHUNK:
            handles[1 - b] = start(c + 1, 1 - b)
        h1, h2 = handles[b]
        h1.wait()
        h2.wait()
        accs = lax.fori_loop(
            0, _GROUPS,
            functools.partial(group_body, lg_bufs[b], gt_bufs[b]),
            accs, unroll=8)

    acc_p, acc_g, acc_pg = accs
    res_v[pl.ds(0, _L)] = acc_p
    res_v[pl.ds(_L, _L)] = acc_g
    res_v[pl.ds(2 * _L, _L)] = acc_pg
    pltpu.sync_copy(res_v, out_hbm.at[pl.ds(wid * 3 * _L, 3 * _L)])


_conf = functools.partial(
    pl.kernel,
    mesh=plsc.VectorSubcoreMesh(core_axis_name="c", subcore_axis_name="s"),
    out_type=jax.ShapeDtypeStruct((_NW * 3 * _L,), jnp.int32),
    compiler_params=pltpu.CompilerParams(needs_layout_passes=False),
    scratch_types=[
        pltpu.VMEM((2 * _CHUNK,), jnp.float32),
        pltpu.VMEM((2 * _CHUNK,), jnp.float32),
        pltpu.VMEM((_CHUNK,), jnp.int32),
        pltpu.VMEM((_CHUNK,), jnp.int32),
        pltpu.VMEM((3 * _L,), jnp.int32),
        pltpu.SemaphoreType.DMA,
        pltpu.SemaphoreType.DMA,
        pltpu.SemaphoreType.DMA,
        pltpu.SemaphoreType.DMA,
    ],
)(_conf_body)


def kernel(pred_logits, gt_labels):
    lg = pred_logits.reshape(-1)
    parts = _conf(lg, gt_labels)
    p = parts.reshape(_NW, 3, _L).sum(axis=(0, 2))
    s_p, s_g, s_pg = p[0], p[1], p[2]
    n = jnp.int32(pred_logits.shape[0])
    c00 = n - s_p - s_g + s_pg
    c01 = s_g - s_pg
    c10 = s_p - s_pg
    c11 = s_pg
    return jnp.stack([jnp.stack([c00, c01]), jnp.stack([c10, c11])]).astype(
        jnp.int32)


# speedup vs baseline: 157.7635x; 156.5165x over previous
import functools

import jax
import jax.numpy as jnp
from jax import lax
from jax.experimental import pallas as pl
from jax.experimental.pallas import tpu as pltpu
from jax.experimental.pallas import tpu_sc as plsc

_NC = 2
_NS = 16
_NW = _NC * _NS
_L = 16

_N = 8388608
_BLK = 128
_NB = _N // _BLK
_BLK_PER_W = _NB // _NW
_BCHUNK = 128
_NCHUNK = _BLK_PER_W // _BCHUNK


def _conf_body(lg_hbm, gt_hbm, out_hbm,
               lg_v0, lg_v1, gt_v0, gt_v1, res_v,
               sem_lg0, sem_lg1, sem_gt0, sem_gt1):
    cid = lax.axis_index("c")
    sid = lax.axis_index("s")
    wid = cid * _NS + sid
    base = wid * _BLK_PER_W

    zeros = jnp.zeros((_L,), jnp.int32)
    ones = jnp.ones((_L,), jnp.int32)

    lg_bufs = (lg_v0, lg_v1)
    gt_bufs = (gt_v0, gt_v1)
    sem_lg = (sem_lg0, sem_lg1)
    sem_gt = (sem_gt0, sem_gt1)

    def start(c, b):
        boff = base + c * _BCHUNK
        h1 = pltpu.async_copy(
            lg_hbm.at[pl.ds(boff, _BCHUNK)], lg_bufs[b], sem_lg[b])
        h2 = pltpu.async_copy(
            gt_hbm.at[pl.ds(boff * _BLK, _BCHUNK * _BLK)], gt_bufs[b],
            sem_gt[b])
        return h1, h2

    def block_body(lg_b, gt_b, blk, accs2):
        a_p, a_g, a_pg = accs2
        for g in range(_BLK // _L):
            l0 = lg_b[blk, 0, pl.ds(g * _L, _L)]
            l1 = lg_b[blk, 1, pl.ds(g * _L, _L)]
            gt16 = gt_b[pl.ds(blk * _BLK + g * _L, _L)]
            pred = l1 > l0
            a_p = a_p + jnp.where(pred, ones, zeros)
            a_g = a_g + gt16
            a_pg = a_pg + jnp.where(pred, gt16, zeros)
        return (a_p, a_g, a_pg)

    z = jnp.zeros((_L,), jnp.int32)
    accs = (z, z, z)
    handles = [None, None]
    handles[0] = start(0, 0)
    for c in range(_NCHUNK):
        b = c % 2
        if c + 1 < _NCHUNK:
            handles[1 - b] = start(c + 1, 1 - b)
        h1, h2 = handles[b]
        h1.wait()
        h2.wait()
        accs = lax.fori_loop(
            0, _BCHUNK,
            functools.partial(block_body, lg_bufs[b], gt_bufs[b]),
            accs)

    acc_p, acc_g, acc_pg = accs
    res_v[pl.ds(0, _L)] = acc_p
    res_v[pl.ds(_L, _L)] = acc_g
    res_v[pl.ds(2 * _L, _L)] = acc_pg
    pltpu.sync_copy(res_v, out_hbm.at[pl.ds(wid * 3 * _L, 3 * _L)])


_conf = functools.partial(
    pl.kernel,
    mesh=plsc.VectorSubcoreMesh(core_axis_name="c", subcore_axis_name="s"),
    out_type=jax.ShapeDtypeStruct((_NW * 3 * _L,), jnp.int32),
    compiler_params=pltpu.CompilerParams(needs_layout_passes=False),
    scratch_types=[
        pltpu.VMEM((_BCHUNK, 2, _BLK), jnp.float32),
        pltpu.VMEM((_BCHUNK, 2, _BLK), jnp.float32),
        pltpu.VMEM((_BCHUNK * _BLK,), jnp.int32),
        pltpu.VMEM((_BCHUNK * _BLK,), jnp.int32),
        pltpu.VMEM((3 * _L,), jnp.int32),
        pltpu.SemaphoreType.DMA,
        pltpu.SemaphoreType.DMA,
        pltpu.SemaphoreType.DMA,
        pltpu.SemaphoreType.DMA,
    ],
)(_conf_body)


def kernel(pred_logits, gt_labels):
    lg = pred_logits.reshape(_NB, _BLK, 2).transpose(0, 2, 1)
    parts = _conf(lg, gt_labels)
    p = parts.reshape(_NW, 3, _L).sum(axis=(0, 2))
    s_p, s_g, s_pg = p[0], p[1], p[2]
    n = jnp.int32(pred_logits.shape[0])
    c00 = n - s_p - s_g + s_pg
    c01 = s_g - s_pg
    c10 = s_p - s_pg
    c11 = s_pg
    return jnp.stack([jnp.stack([c00, c01]), jnp.stack([c10, c11])]).astype(
        jnp.int32)


# speedup vs baseline: 167.3076x; 1.0605x over previous
import functools

import jax
import jax.numpy as jnp
from jax import lax
from jax.experimental import pallas as pl
from jax.experimental.pallas import tpu as pltpu
from jax.experimental.pallas import tpu_sc as plsc

_NC = 2
_NS = 16
_NW = _NC * _NS
_L = 16

_N = 8388608
_BLK = 128
_NB = _N // _BLK
_BLK_PER_W = _NB // _NW
_BCHUNK = 128
_NCHUNK = _BLK_PER_W // _BCHUNK


def _conf_body(lg_hbm, gt_hbm, out_hbm,
               lg_v0, lg_v1, gt_v0, gt_v1, res_v,
               sem_lg0, sem_lg1, sem_gt0, sem_gt1):
    cid = lax.axis_index("c")
    sid = lax.axis_index("s")
    wid = cid * _NS + sid
    base = wid * _BLK_PER_W

    zeros = jnp.zeros((_L,), jnp.int32)
    ones = jnp.ones((_L,), jnp.int32)

    lg_bufs = (lg_v0, lg_v1)
    gt_bufs = (gt_v0, gt_v1)
    sem_lg = (sem_lg0, sem_lg1)
    sem_gt = (sem_gt0, sem_gt1)

    def start(c, b):
        boff = base + c * _BCHUNK
        h1 = pltpu.async_copy(
            lg_hbm.at[pl.ds(boff, _BCHUNK)], lg_bufs[b], sem_lg[b])
        h2 = pltpu.async_copy(
            gt_hbm.at[pl.ds(boff * _BLK, _BCHUNK * _BLK)], gt_bufs[b],
            sem_gt[b])
        return h1, h2

    def block_body(lg_b, gt_b, blk, accs2):
        a_p, a_g, a_pg = accs2
        for g in range(_BLK // _L):
            l0 = lg_b[blk, 0, pl.ds(g * _L, _L)]
            l1 = lg_b[blk, 1, pl.ds(g * _L, _L)]
            gt16 = gt_b[pl.ds(blk * _BLK + g * _L, _L)]
            pred = l1 > l0
            a_p = a_p + jnp.where(pred, ones, zeros)
            a_g = a_g + gt16
            a_pg = a_pg + jnp.where(pred, gt16, zeros)
        return (a_p, a_g, a_pg)

    z = jnp.zeros((_L,), jnp.int32)
    accs = (z, z, z)
    handles = [None, None]
    handles[0] = start(0, 0)
    for c in range(_NCHUNK):
        b = c % 2
        if c + 1 < _NCHUNK:
            handles[1 - b] = start(c + 1, 1 - b)
        h1, h2 = handles[b]
        h1.wait()
        h2.wait()
        accs = lax.fori_loop(
            0, _BCHUNK,
            functools.partial(block_body, lg_bufs[b], gt_bufs[b]),
            accs, unroll=2)

    acc_p, acc_g, acc_pg = accs
    res_v[pl.ds(0, _L)] = acc_p
    res_v[pl.ds(_L, _L)] = acc_g
    res_v[pl.ds(2 * _L, _L)] = acc_pg
    pltpu.sync_copy(res_v, out_hbm.at[pl.ds(wid * 3 * _L, 3 * _L)])


_conf = functools.partial(
    pl.kernel,
    mesh=plsc.VectorSubcoreMesh(core_axis_name="c", subcore_axis_name="s"),
    out_type=jax.ShapeDtypeStruct((_NW * 3 * _L,), jnp.int32),
    compiler_params=pltpu.CompilerParams(needs_layout_passes=False),
    scratch_types=[
        pltpu.VMEM((_BCHUNK, 2, _BLK), jnp.float32),
        pltpu.VMEM((_BCHUNK, 2, _BLK), jnp.float32),
        pltpu.VMEM((_BCHUNK * _BLK,), jnp.int32),
        pltpu.VMEM((_BCHUNK * _BLK,), jnp.int32),
        pltpu.VMEM((3 * _L,), jnp.int32),
        pltpu.SemaphoreType.DMA,
        pltpu.SemaphoreType.DMA,
        pltpu.SemaphoreType.DMA,
        pltpu.SemaphoreType.DMA,
    ],
)(_conf_body)


def kernel(pred_logits, gt_labels):
    lg = pred_logits.reshape(_NB, _BLK, 2).transpose(0, 2, 1)
    parts = _conf(lg, gt_labels)
    s = parts.reshape(_NW, 3, _L).sum(axis=(0, 2))
    mix = jnp.array([[-1, -1, 1], [0, 1, -1], [1, 0, -1], [0, 0, 1]],
                    dtype=jnp.int32)
    off = jnp.array([pred_logits.shape[0], 0, 0, 0], dtype=jnp.int32)
    return (mix @ s + off).reshape(2, 2)


# speedup vs baseline: 171.4487x; 1.0248x over previous
import functools

import jax
import jax.numpy as jnp
from jax import lax
from jax.experimental import pallas as pl
from jax.experimental.pallas import tpu as pltpu
from jax.experimental.pallas import tpu_sc as plsc

_NC = 2
_NS = 16
_NW = _NC * _NS
_L = 16

_N = 8388608
_BLK = 128
_NB = _N // _BLK
_BLK_PER_W = _NB // _NW
_BCHUNK = 64
_NCHUNK = _BLK_PER_W // _BCHUNK
_NBUF = 4


def _conf_body(lg_hbm, gt_hbm, out_hbm, *scratch):
    lg_bufs = scratch[0:_NBUF]
    gt_bufs = scratch[_NBUF:2 * _NBUF]
    res_v = scratch[2 * _NBUF]
    sem_lg = scratch[2 * _NBUF + 1:3 * _NBUF + 1]
    sem_gt = scratch[3 * _NBUF + 1:4 * _NBUF + 1]

    cid = lax.axis_index("c")
    sid = lax.axis_index("s")
    wid = cid * _NS + sid
    base = wid * _BLK_PER_W

    zeros = jnp.zeros((_L,), jnp.int32)
    ones = jnp.ones((_L,), jnp.int32)

    def start(c, b):
        boff = base + c * _BCHUNK
        h1 = pltpu.async_copy(
            lg_hbm.at[pl.ds(boff, _BCHUNK)], lg_bufs[b], sem_lg[b])
        h2 = pltpu.async_copy(
            gt_hbm.at[pl.ds(boff * _BLK, _BCHUNK * _BLK)], gt_bufs[b],
            sem_gt[b])
        return h1, h2

    def block_body(lg_b, gt_b, blk, accs2):
        a_p, a_g, a_pg = accs2
        for g in range(_BLK // _L):
            l0 = lg_b[blk, 0, pl.ds(g * _L, _L)]
            l1 = lg_b[blk, 1, pl.ds(g * _L, _L)]
            gt16 = gt_b[pl.ds(blk * _BLK + g * _L, _L)]
            pred = l1 > l0
            a_p = a_p + jnp.where(pred, ones, zeros)
            a_g = a_g + gt16
            a_pg = a_pg + jnp.where(pred, gt16, zeros)
        return (a_p, a_g, a_pg)

    z = jnp.zeros((_L,), jnp.int32)
    accs = (z, z, z)
    handles = [None] * _NBUF
    for c in range(_NBUF - 1):
        handles[c] = start(c, c)
    for c in range(_NCHUNK):
        b = c % _NBUF
        nxt = c + _NBUF - 1
        if nxt < _NCHUNK:
            handles[nxt % _NBUF] = start(nxt, nxt % _NBUF)
        h1, h2 = handles[b]
        h1.wait()
        h2.wait()
        accs = lax.fori_loop(
            0, _BCHUNK,
            functools.partial(block_body, lg_bufs[b], gt_bufs[b]),
            accs, unroll=2)

    acc_p, acc_g, acc_pg = accs
    res_v[pl.ds(0, _L)] = acc_p
    res_v[pl.ds(_L, _L)] = acc_g
    res_v[pl.ds(2 * _L, _L)] = acc_pg
    pltpu.sync_copy(res_v, out_hbm.at[pl.ds(wid * 3 * _L, 3 * _L)])


_conf = functools.partial(
    pl.kernel,
    mesh=plsc.VectorSubcoreMesh(core_axis_name="c", subcore_axis_name="s"),
    out_type=jax.ShapeDtypeStruct((_NW * 3 * _L,), jnp.int32),
    compiler_params=pltpu.CompilerParams(needs_layout_passes=False),
    scratch_types=(
        [pltpu.VMEM((_BCHUNK, 2, _BLK), jnp.float32)] * _NBUF
        + [pltpu.VMEM((_BCHUNK * _BLK,), jnp.int32)] * _NBUF
        + [pltpu.VMEM((3 * _L,), jnp.int32)]
        + [pltpu.SemaphoreType.DMA] * (2 * _NBUF)
    ),
)(_conf_body)


def kernel(pred_logits, gt_labels):
    lg = pred_logits.reshape(_NB, _BLK, 2).transpose(0, 2, 1)
    parts = _conf(lg, gt_labels)
    s = parts.reshape(_NW, 3, _L).sum(axis=(0, 2))
    mix = jnp.array([[-1, -1, 1], [0, 1, -1], [1, 0, -1], [0, 0, 1]],
                    dtype=jnp.int32)
    off = jnp.array([pred_logits.shape[0], 0, 0, 0], dtype=jnp.int32)
    return (mix @ s + off).reshape(2, 2)


# speedup vs baseline: 195.1114x; 1.1380x over previous
import functools

import jax
import jax.numpy as jnp
from jax import lax
from jax.experimental import pallas as pl
from jax.experimental.pallas import tpu as pltpu
from jax.experimental.pallas import tpu_sc as plsc

_NC = 2
_NS = 16
_NW = _NC * _NS
_L = 16

_N = 8388608
_BLK = 128
_NB = _N // _BLK
_BLK_PER_W = _NB // _NW
_BCHUNK = 64
_NCHUNK = _BLK_PER_W // _BCHUNK


def _conf_body(lg_hbm, gt_hbm, out_hbm,
               lg_v0, lg_v1, gt_v0, gt_v1, res_v,
               sem_lg0, sem_lg1, sem_gt0, sem_gt1):
    cid = lax.axis_index("c")
    sid = lax.axis_index("s")
    wid = cid * _NS + sid
    base = wid * _BLK_PER_W

    zeros = jnp.zeros((_L,), jnp.int32)
    ones = jnp.ones((_L,), jnp.int32)

    lg_bufs = (lg_v0, lg_v1)
    gt_bufs = (gt_v0, gt_v1)
    sem_lg = (sem_lg0, sem_lg1)
    sem_gt = (sem_gt0, sem_gt1)

    def copies(c, b):
        boff = base + c * _BCHUNK
        h1 = pltpu.make_async_copy(
            lg_hbm.at[pl.ds(boff, _BCHUNK)], lg_bufs[b], sem_lg[b])
        h2 = pltpu.make_async_copy(
            gt_hbm.at[pl.ds(boff * _BLK, _BCHUNK * _BLK)], gt_bufs[b],
            sem_gt[b])
        return h1, h2

    def block_body(lg_b, gt_b, blk, accs2):
        a_p, a_g, a_pg = accs2
        for g in range(_BLK // _L):
            l0 = lg_b[blk, 0, pl.ds(g * _L, _L)]
            l1 = lg_b[blk, 1, pl.ds(g * _L, _L)]
            gt16 = gt_b[pl.ds(blk * _BLK + g * _L, _L)]
            pred = l1 > l0
            a_p = a_p + jnp.where(pred, ones, zeros)
            a_g = a_g + gt16
            a_pg = a_pg + jnp.where(pred, gt16, zeros)
        return (a_p, a_g, a_pg)

    for b in range(2):
        h1, h2 = copies(b, b)
        h1.start()
        h2.start()

    def pair_body(c2, accs):
        for b in range(2):
            c = c2 * 2 + b
            h1, h2 = copies(c, b)
            h1.wait()
            h2.wait()

            @pl.when(c + 2 < _NCHUNK)
            def _():
                n1, n2 = copies(c + 2, b)
                n1.start()
                n2.start()

            accs = lax.fori_loop(
                0, _BCHUNK,
                functools.partial(block_body, lg_bufs[b], gt_bufs[b]),
                accs, unroll=2)
        return accs

    z = jnp.zeros((_L,), jnp.int32)
    acc_p, acc_g, acc_pg = lax.fori_loop(
        0, _NCHUNK // 2, pair_body, (z, z, z))
    res_v[pl.ds(0, _L)] = acc_p
    res_v[pl.ds(_L, _L)] = acc_g
    res_v[pl.ds(2 * _L, _L)] = acc_pg
    pltpu.sync_copy(res_v, out_hbm.at[pl.ds(wid * 3 * _L, 3 * _L)])


_conf = functools.partial(
    pl.kernel,
    mesh=plsc.VectorSubcoreMesh(core_axis_name="c", subcore_axis_name="s"),
    out_type=jax.ShapeDtypeStruct((_NW * 3 * _L,), jnp.int32),
    compiler_params=pltpu.CompilerParams(needs_layout_passes=False),
    scratch_types=(
        [pltpu.VMEM((_BCHUNK, 2, _BLK), jnp.float32)] * 2
        + [pltpu.VMEM((_BCHUNK * _BLK,), jnp.int32)] * 2
        + [pltpu.VMEM((3 * _L,), jnp.int32)]
        + [pltpu.SemaphoreType.DMA] * 4
    ),
)(_conf_body)


def kernel(pred_logits, gt_labels):
    lg = pred_logits.reshape(_NB, _BLK, 2).transpose(0, 2, 1)
    parts = _conf(lg, gt_labels)
    s = parts.reshape(_NW, 3, _L).sum(axis=(0, 2))
    mix = jnp.array([[-1, -1, 1], [0, 1, -1], [1, 0, -1], [0, 0, 1]],
                    dtype=jnp.int32)
    off = jnp.array([pred_logits.shape[0], 0, 0, 0], dtype=jnp.int32)
    return (mix @ s + off).reshape(2, 2)
